# baseline (speedup 1.0000x reference)
import functools

import jax
import jax.numpy as jnp
from jax import lax
from jax.experimental import pallas as pl
from jax.experimental.pallas import tpu as pltpu
from jax.experimental.pallas import tpu_sc as plsc

N = 10000
E = 320000
G = 32
H = 8

NC = 2
NS = 16
KK = 80
NCH = 132
EPW = KK * NCH
EPAD = EPW * NC * NS
NW = NC * NS
NR = 10016
RPT = NR // NS


def _combine_body(hc_in, raw_ref, b_ref, xin_ref):
    raw = raw_ref[...]
    acc = raw[0] + raw[1]
    nb = acc.shape[0]
    num = acc[:, :hc_in].reshape(nb, H, hc_in // H)
    den = acc[:, hc_in:hc_in + 8]
    xin = num / (den[:, :, None] + 1e-16)
    xin = xin.reshape(nb, hc_in) + b_ref[...][None, :]
    xin_ref[...] = jnp.where(xin > 0, xin, jnp.exp(xin) - 1.0)


@functools.partial(jax.jit, static_argnames=("hc_in",))
def _combine(raw, b, hc_in):
    row_in = hc_in + 16
    nrb = NR // 4
    return pl.pallas_call(
        functools.partial(_combine_body, hc_in),
        grid=(4,),
        in_specs=[
            pl.BlockSpec((2, nrb, row_in), lambda i: (0, i, 0)),
            pl.BlockSpec((hc_in,), lambda i: (0,)),
        ],
        out_specs=pl.BlockSpec((nrb, hc_in), lambda i: (i, 0)),
        out_shape=jax.ShapeDtypeStruct((NR, hc_in), jnp.float32),
    )(raw, b)


def _dense_body0(hc, x_ref, w_ref, as_ref, ad_ref, hext_ref, dstt_ref):
    n = x_ref.shape[0]
    h = jnp.dot(x_ref[...], w_ref[...], preferred_element_type=jnp.float32)
    asrc = jnp.dot(h, as_ref[...], preferred_element_type=jnp.float32)
    adst = jnp.dot(h, ad_ref[...], preferred_element_type=jnp.float32)
    hext_ref[...] = jnp.concatenate(
        [h, asrc, jnp.zeros((n, 8), jnp.float32)], axis=1)
    m = asrc.max(axis=0)[None, :] + adst
    mub = jnp.where(m > 0, m, 0.2 * m)
    dstt_ref[...] = jnp.concatenate([adst, mub], axis=1)


@functools.partial(jax.jit, static_argnames=("hc",))
def _dense0(x, W, As, Ad, hc):
    n = x.shape[0]
    row = hc + 16
    return pl.pallas_call(
        functools.partial(_dense_body0, hc),
        out_shape=(
            jax.ShapeDtypeStruct((n, row), jnp.float32),
            jax.ShapeDtypeStruct((n, 16), jnp.float32),
        ),
    )(x, W, As, Ad)


def _head_body(h_ref, batch_ref, f1w_ref, f1b_ref, f2w_ref, f2b_ref, out_ref):
    h = h_ref[...]
    gid = lax.broadcasted_iota(jnp.int32, (N, G), 1)
    P = (batch_ref[...] == gid).astype(jnp.float32)
    sums = lax.dot_general(P, h, (((0,), (0,)), ((), ())),
                           preferred_element_type=jnp.float32)
    cnt = jnp.sum(P, axis=0)
    gm = sums / jnp.maximum(cnt, 1.0)[:, None]
    z = jnp.dot(gm, f1w_ref[...], preferred_element_type=jnp.float32)
    z = jnp.maximum(z + f1b_ref[...][None, :], 0.0)
    out_ref[...] = (jnp.dot(z, f2w_ref[...], preferred_element_type=jnp.float32)
                    + f2b_ref[...][None, :])


@jax.jit
def _head(h, batch2, f1w, f1b, f2w, f2b):
    return pl.pallas_call(
        _head_body,
        out_shape=jax.ShapeDtypeStruct((G, 1), jnp.float32),
    )(h, batch2, f1w, f1b, f2w, f2b)


def _att_mat(a):
    Hh, C = a.shape
    return (jnp.eye(Hh, dtype=a.dtype)[:, None, :] * a[:, :, None]).reshape(Hh * C, Hh)


def _sc_edge_body(hc, idx_all, hext, dstt, out,
                  I0, I1, I2, I3, S0, Dt0, S1, Dt1, R0, accum,
                  sS0, sD0, sS1, sD1, sI0, sI1, sI2, sI3, sW):
    row = hc + 16
    nv = hc // 16
    c = lax.axis_index("c")
    s = lax.axis_index("s")
    wid = c * NS + s

    cph = hc // H
    lanes = lax.iota(jnp.int32, 16)
    idx_hi = lanes % 8 + 8
    mask_den = jnp.where(lanes < 8, 1.0, 0.0).astype(jnp.float32)

    Islots = (I0, I1, I2, I3)
    Isems = (sI0, sI1, sI2, sI3)
    data = ((S0, Dt0, R0, sS0, sD0), (S1, Dt1, R0, sS1, sD1))

    def _zrow(k, _):
        for j in range(row // 16):
            R0[k, pl.ds(16 * j, 16)] = jnp.zeros((16,), jnp.float32)
        return 0
    lax.fori_loop(0, KK, _zrow, 0)
    chunks = [KK] * (RPT // KK) + ([RPT % KK] if RPT % KK else [])
    off = 0
    for sz in chunks:
        pltpu.async_copy(R0.at[pl.ds(0, sz)],
                         accum.at[pl.ds(s * RPT + off, sz)], sW)
        off += sz
    off = 0
    for sz in chunks:
        pltpu.make_async_copy(R0.at[pl.ds(0, sz)],
                              accum.at[pl.ds(s * RPT + off, sz)], sW).wait()
        off += sz
    plsc.subcore_barrier()

    def _issue_idx(isl, g):
        pltpu.async_copy(idx_all.at[wid, g], Islots[isl], Isems[isl])

    def _issue_gather(isl, b):
        S, Dt, _, sS, sD = data[b]
        pltpu.async_copy(hext.at[Islots[isl].at[0]], S, sS)
        pltpu.async_copy(dstt.at[Islots[isl].at[1]], Dt, sD)

    for g in range(4):
        _issue_idx(g, g)
    for g in range(2):
        pltpu.make_async_copy(idx_all.at[wid, g], Islots[g], Isems[g]).wait()
        _issue_gather(g, g)

    def _quad(gq, _):
        for b4 in range(4):
            g = 4 * gq + b4
            b = b4 % 2
            isl = b4
            S, Dt, R, sS, sD = data[b]
            pltpu.make_async_copy(hext.at[Islots[isl].at[0]], S, sS).wait()
            pltpu.make_async_copy(dstt.at[Islots[isl].at[1]], Dt, sD).wait()

            @plsc.parallel_loop(0, KK, unroll=2)
            def _edge(k):
                sa = S[k, pl.ds(hc, 16)]
                dv = Dt[k, pl.ds(0, 16)]
                e16 = sa + dv
                lr = jnp.where(e16 > 0, e16, 0.2 * e16)
                mub16 = dv.at[idx_hi].get(mode="promise_in_bounds")
                wv = jnp.exp(lr - mub16)
                for j in range(nv):
                    if cph == 16:
                        idxj = lanes * 0 + j
                    else:
                        idxj = jnp.where(lanes < 8, 2 * j, 2 * j + 1)
                    wj = wv.at[idxj].get(mode="promise_in_bounds")
                    R[k, pl.ds(16 * j, 16)] = wj * S[k, pl.ds(16 * j, 16)]
                R[k, pl.ds(hc, 16)] = wv * mask_den

            pltpu.sync_copy(R, accum.at[Islots[isl].at[1]], add=True)

            @pl.when(g + 2 < NCH)
            def _():
                isl2 = (b4 + 2) % 4
                pltpu.make_async_copy(
                    idx_all.at[wid, g + 2], Islots[isl2], Isems[isl2]).wait()
                _issue_gather(isl2, b)

            @pl.when(g + 4 < NCH)
            def _():
                _issue_idx(isl, g + 4)
        return 0

    lax.fori_loop(0, NCH // 4, _quad, 0)
    plsc.subcore_barrier()

    off = 0
    for sz in chunks:
        r0 = s * RPT + off
        pltpu.async_copy(accum.at[pl.ds(r0, sz)], out.at[c, pl.ds(r0, sz)], sW)
        off += sz
    off = 0
    for sz in chunks:
        r0 = s * RPT + off
        pltpu.make_async_copy(accum.at[pl.ds(r0, sz)],
                              out.at[c, pl.ds(r0, sz)], sW).wait()
        off += sz


@functools.lru_cache(maxsize=None)
def _sc_edge_fn(hc):
    row = hc + 16
    mesh = plsc.VectorSubcoreMesh(core_axis_name="c", subcore_axis_name="s",
                                  num_cores=NC, num_subcores=NS)
    return pl.kernel(
        functools.partial(_sc_edge_body, hc),
        out_type=jax.ShapeDtypeStruct((NC, NR, row), jnp.float32),
        mesh=mesh,
        compiler_params=pltpu.CompilerParams(use_tc_tiling_on_sc=False),
        scratch_types=[
            pltpu.VMEM((2, KK), jnp.int32),
            pltpu.VMEM((2, KK), jnp.int32),
            pltpu.VMEM((2, KK), jnp.int32),
            pltpu.VMEM((2, KK), jnp.int32),
            pltpu.VMEM((KK, row), jnp.float32),
            pltpu.VMEM((KK, 16), jnp.float32),
            pltpu.VMEM((KK, row), jnp.float32),
            pltpu.VMEM((KK, 16), jnp.float32),
            pltpu.VMEM((KK, row), jnp.float32),
            pltpu.VMEM_SHARED((NR, row), jnp.float32),
            pltpu.SemaphoreType.DMA,
            pltpu.SemaphoreType.DMA,
            pltpu.SemaphoreType.DMA,
            pltpu.SemaphoreType.DMA,
            pltpu.SemaphoreType.DMA,
            pltpu.SemaphoreType.DMA,
            pltpu.SemaphoreType.DMA,
            pltpu.SemaphoreType.DMA,
            pltpu.SemaphoreType.DMA,
        ],
    )


def kernel(x, edge_index, batch, W1, a1s, a1d, b1, W2, a2s, a2d, b2,
           W3, a3s, a3d, b3, fc1_w, fc1_b, fc2_w, fc2_b):
    loop = jnp.arange(N, dtype=jnp.int32)
    fill = jnp.full((EPAD - E - N,), N, jnp.int32)
    src_pad = jnp.concatenate([edge_index[0].astype(jnp.int32), loop, fill])
    dst_pad = jnp.concatenate([edge_index[1].astype(jnp.int32), loop, fill])
    idx_all = jnp.stack([src_pad.reshape(NW, NCH, KK),
                         dst_pad.reshape(NW, NCH, KK)], axis=2)

    def tables(hext, dstt):
        hext = jnp.concatenate(
            [hext, jnp.zeros((1, hext.shape[1]), jnp.float32)], axis=0)
        dstt = jnp.concatenate([dstt, jnp.zeros((1, 16), jnp.float32)], axis=0)
        return hext, dstt

    hext, dstt = _dense0(x, W1, _att_mat(a1s), _att_mat(a1d), 64)
    raw = _sc_edge_fn(64)(idx_all, *tables(hext, dstt))
    xin = _combine(raw, b1, 64)
    hext, dstt = _dense0(xin, W2, _att_mat(a2s), _att_mat(a2d), 128)
    raw = _sc_edge_fn(128)(idx_all, hext, dstt)
    xin = _combine(raw, b2, 128)
    hext, dstt = _dense0(xin, W3, _att_mat(a3s), _att_mat(a3d), 128)
    raw = _sc_edge_fn(128)(idx_all, hext, dstt)
    h3 = _combine(raw, b3, 128)[:N]

    batch2 = batch.astype(jnp.int32).reshape(N, 1)
    return _head(h3, batch2, fc1_w, fc1_b, fc2_w, fc2_b)


# speedup vs baseline: 1.1696x; 1.1696x over previous
import functools

import jax
import jax.numpy as jnp
from jax import lax
from jax.experimental import pallas as pl
from jax.experimental.pallas import tpu as pltpu
from jax.experimental.pallas import tpu_sc as plsc

N = 10000
E = 320000
G = 32
H = 8

NC = 2
NS = 16
KK = 64
NCH = 164
EPW = KK * NCH
EPAD = EPW * NC * NS
NW = NC * NS
NR = 10016
RPT = NR // NS


def _combine_body(hc_in, raw_ref, b_ref, xin_ref):
    raw = raw_ref[...]
    acc = raw[0] + raw[1]
    nb = acc.shape[0]
    num = acc[:, :hc_in].reshape(nb, H, hc_in // H)
    den = acc[:, hc_in:hc_in + 8]
    xin = num / (den[:, :, None] + 1e-16)
    xin = xin.reshape(nb, hc_in) + b_ref[...][None, :]
    xin_ref[...] = jnp.where(xin > 0, xin, jnp.exp(xin) - 1.0)


@functools.partial(jax.jit, static_argnames=("hc_in",))
def _combine(raw, b, hc_in):
    row_in = hc_in + 16
    nrb = NR // 4
    return pl.pallas_call(
        functools.partial(_combine_body, hc_in),
        grid=(4,),
        in_specs=[
            pl.BlockSpec((2, nrb, row_in), lambda i: (0, i, 0)),
            pl.BlockSpec((hc_in,), lambda i: (0,)),
        ],
        out_specs=pl.BlockSpec((nrb, hc_in), lambda i: (i, 0)),
        out_shape=jax.ShapeDtypeStruct((NR, hc_in), jnp.float32),
    )(raw, b)


def _dense_body0(hc, x_ref, w_ref, as_ref, ad_ref, hext_ref, dstt_ref):
    n = x_ref.shape[0]
    h = jnp.dot(x_ref[...], w_ref[...], preferred_element_type=jnp.float32)
    asrc = jnp.dot(h, as_ref[...], preferred_element_type=jnp.float32)
    adst = jnp.dot(h, ad_ref[...], preferred_element_type=jnp.float32)
    hext_ref[...] = jnp.concatenate(
        [h, asrc, jnp.zeros((n, 8), jnp.float32)], axis=1)
    m = asrc.max(axis=0)[None, :] + adst
    mub = jnp.where(m > 0, m, 0.2 * m)
    dstt_ref[...] = jnp.concatenate([adst, mub], axis=1)


@functools.partial(jax.jit, static_argnames=("hc",))
def _dense0(x, W, As, Ad, hc):
    n = x.shape[0]
    row = hc + 16
    return pl.pallas_call(
        functools.partial(_dense_body0, hc),
        out_shape=(
            jax.ShapeDtypeStruct((n, row), jnp.float32),
            jax.ShapeDtypeStruct((n, 16), jnp.float32),
        ),
    )(x, W, As, Ad)


def _head_body(h_ref, batch_ref, f1w_ref, f1b_ref, f2w_ref, f2b_ref, out_ref):
    h = h_ref[...]
    gid = lax.broadcasted_iota(jnp.int32, (N, G), 1)
    P = (batch_ref[...] == gid).astype(jnp.float32)
    sums = lax.dot_general(P, h, (((0,), (0,)), ((), ())),
                           preferred_element_type=jnp.float32)
    cnt = jnp.sum(P, axis=0)
    gm = sums / jnp.maximum(cnt, 1.0)[:, None]
    z = jnp.dot(gm, f1w_ref[...], preferred_element_type=jnp.float32)
    z = jnp.maximum(z + f1b_ref[...][None, :], 0.0)
    out_ref[...] = (jnp.dot(z, f2w_ref[...], preferred_element_type=jnp.float32)
                    + f2b_ref[...][None, :])


@jax.jit
def _head(h, batch2, f1w, f1b, f2w, f2b):
    return pl.pallas_call(
        _head_body,
        out_shape=jax.ShapeDtypeStruct((G, 1), jnp.float32),
    )(h, batch2, f1w, f1b, f2w, f2b)


def _att_mat(a):
    Hh, C = a.shape
    return (jnp.eye(Hh, dtype=a.dtype)[:, None, :] * a[:, :, None]).reshape(Hh * C, Hh)


def _sc_edge_body(hc, idx_all, hext, dstt, out,
                  I0, I1, I2, I3, S0, Dt0, R0, S1, Dt1, R1, accum,
                  sS0, sD0, sS1, sD1, sI0, sI1, sI2, sI3, sW):
    row = hc + 16
    nv = hc // 16
    c = lax.axis_index("c")
    s = lax.axis_index("s")
    wid = c * NS + s

    cph = hc // H
    lanes = lax.iota(jnp.int32, 16)
    idx_hi = lanes % 8 + 8
    mask_den = jnp.where(lanes < 8, 1.0, 0.0).astype(jnp.float32)

    Islots = (I0, I1, I2, I3)
    Isems = (sI0, sI1, sI2, sI3)
    data = ((S0, Dt0, R0, sS0, sD0), (S1, Dt1, R1, sS1, sD1))

    def _zrow(k, _):
        for j in range(row // 16):
            R0[k, pl.ds(16 * j, 16)] = jnp.zeros((16,), jnp.float32)
        return 0
    lax.fori_loop(0, KK, _zrow, 0)
    chunks = [KK] * (RPT // KK) + ([RPT % KK] if RPT % KK else [])
    off = 0
    for sz in chunks:
        pltpu.async_copy(R0.at[pl.ds(0, sz)],
                         accum.at[pl.ds(s * RPT + off, sz)], sW)
        off += sz
    off = 0
    for sz in chunks:
        pltpu.make_async_copy(R0.at[pl.ds(0, sz)],
                              accum.at[pl.ds(s * RPT + off, sz)], sW).wait()
        off += sz
    plsc.subcore_barrier()

    def _issue_idx(isl, g):
        pltpu.async_copy(idx_all.at[wid, g], Islots[isl], Isems[isl])

    def _issue_gather(isl, b):
        S, Dt, _, sS, sD = data[b]
        pltpu.async_copy(hext.at[Islots[isl].at[0]], S, sS)
        pltpu.async_copy(dstt.at[Islots[isl].at[1]], Dt, sD)

    for g in range(4):
        _issue_idx(g, g)
    for g in range(2):
        pltpu.make_async_copy(idx_all.at[wid, g], Islots[g], Isems[g]).wait()
        _issue_gather(g, g)

    def _quad(gq, _):
        for b4 in range(4):
            g = 4 * gq + b4
            b = b4 % 2
            isl = b4
            S, Dt, R, sS, sD = data[b]
            pltpu.make_async_copy(hext.at[Islots[isl].at[0]], S, sS).wait()
            pltpu.make_async_copy(dstt.at[Islots[isl].at[1]], Dt, sD).wait()

            @plsc.parallel_loop(0, KK, unroll=2)
            def _edge(k):
                sa = S[k, pl.ds(hc, 16)]
                dv = Dt[k, pl.ds(0, 16)]
                e16 = sa + dv
                lr = jnp.where(e16 > 0, e16, 0.2 * e16)
                mub16 = dv.at[idx_hi].get(mode="promise_in_bounds")
                wv = jnp.exp(lr - mub16)
                for j in range(nv):
                    if cph == 16:
                        idxj = lanes * 0 + j
                    else:
                        idxj = jnp.where(lanes < 8, 2 * j, 2 * j + 1)
                    wj = wv.at[idxj].get(mode="promise_in_bounds")
                    R[k, pl.ds(16 * j, 16)] = wj * S[k, pl.ds(16 * j, 16)]
                R[k, pl.ds(hc, 16)] = wv * mask_den

            pltpu.sync_copy(R, accum.at[Islots[isl].at[1]], add=True)

            @pl.when(g + 2 < NCH)
            def _():
                isl2 = (b4 + 2) % 4
                pltpu.make_async_copy(
                    idx_all.at[wid, g + 2], Islots[isl2], Isems[isl2]).wait()
                _issue_gather(isl2, b)

            @pl.when(g + 4 < NCH)
            def _():
                _issue_idx(isl, g + 4)
        return 0

    lax.fori_loop(0, NCH // 4, _quad, 0)
    plsc.subcore_barrier()

    off = 0
    for sz in chunks:
        r0 = s * RPT + off
        pltpu.async_copy(accum.at[pl.ds(r0, sz)], out.at[c, pl.ds(r0, sz)], sW)
        off += sz
    off = 0
    for sz in chunks:
        r0 = s * RPT + off
        pltpu.make_async_copy(accum.at[pl.ds(r0, sz)],
                              out.at[c, pl.ds(r0, sz)], sW).wait()
        off += sz


@functools.lru_cache(maxsize=None)
def _sc_edge_fn(hc):
    row = hc + 16
    mesh = plsc.VectorSubcoreMesh(core_axis_name="c", subcore_axis_name="s",
                                  num_cores=NC, num_subcores=NS)
    return pl.kernel(
        functools.partial(_sc_edge_body, hc),
        out_type=jax.ShapeDtypeStruct((NC, NR, row), jnp.float32),
        mesh=mesh,
        compiler_params=pltpu.CompilerParams(use_tc_tiling_on_sc=False),
        scratch_types=[
            pltpu.VMEM((2, KK), jnp.int32),
            pltpu.VMEM((2, KK), jnp.int32),
            pltpu.VMEM((2, KK), jnp.int32),
            pltpu.VMEM((2, KK), jnp.int32),
            pltpu.VMEM((KK, row), jnp.float32),
            pltpu.VMEM((KK, 16), jnp.float32),
            pltpu.VMEM((KK, row), jnp.float32),
            pltpu.VMEM((KK, row), jnp.float32),
            pltpu.VMEM((KK, 16), jnp.float32),
            pltpu.VMEM((KK, row), jnp.float32),
            pltpu.VMEM_SHARED((NR, row), jnp.float32),
            pltpu.SemaphoreType.DMA,
            pltpu.SemaphoreType.DMA,
            pltpu.SemaphoreType.DMA,
            pltpu.SemaphoreType.DMA,
            pltpu.SemaphoreType.DMA,
            pltpu.SemaphoreType.DMA,
            pltpu.SemaphoreType.DMA,
            pltpu.SemaphoreType.DMA,
            pltpu.SemaphoreType.DMA,
        ],
    )


def kernel(x, edge_index, batch, W1, a1s, a1d, b1, W2, a2s, a2d, b2,
           W3, a3s, a3d, b3, fc1_w, fc1_b, fc2_w, fc2_b):
    loop = jnp.arange(N, dtype=jnp.int32)
    fill = jnp.full((EPAD - E - N,), N, jnp.int32)
    src_pad = jnp.concatenate([edge_index[0].astype(jnp.int32), loop, fill])
    dst_pad = jnp.concatenate([edge_index[1].astype(jnp.int32), loop, fill])
    idx_all = jnp.stack([src_pad.reshape(NW, NCH, KK),
                         dst_pad.reshape(NW, NCH, KK)], axis=2)

    def tables(hext, dstt):
        hext = jnp.concatenate(
            [hext, jnp.zeros((1, hext.shape[1]), jnp.float32)], axis=0)
        dstt = jnp.concatenate([dstt, jnp.zeros((1, 16), jnp.float32)], axis=0)
        return hext, dstt

    hext, dstt = _dense0(x, W1, _att_mat(a1s), _att_mat(a1d), 64)
    raw = _sc_edge_fn(64)(idx_all, *tables(hext, dstt))
    xin = _combine(raw, b1, 64)
    hext, dstt = _dense0(xin, W2, _att_mat(a2s), _att_mat(a2d), 128)
    raw = _sc_edge_fn(128)(idx_all, hext, dstt)
    xin = _combine(raw, b2, 128)
    hext, dstt = _dense0(xin, W3, _att_mat(a3s), _att_mat(a3d), 128)
    raw = _sc_edge_fn(128)(idx_all, hext, dstt)
    h3 = _combine(raw, b3, 128)[:N]

    batch2 = batch.astype(jnp.int32).reshape(N, 1)
    return _head(h3, batch2, fc1_w, fc1_b, fc2_w, fc2_b)


# speedup vs baseline: 1.7062x; 1.4588x over previous
import functools

import jax
import jax.numpy as jnp
from jax import lax
from jax.experimental import pallas as pl
from jax.experimental.pallas import tpu as pltpu
from jax.experimental.pallas import tpu_sc as plsc

N = 10000
E = 320000
G = 32
H = 8

NC = 2
NS = 16
KK = 64
NCH_A = 216
NCH_B = 108
TOTCH = NS * (NCH_A + NCH_B)
EPAD = TOTCH * KK
NW = NC * NS
NR = 10016
RPT = NR // NS


def _combine_body(hc_in, raw_ref, b_ref, xin_ref):
    raw = raw_ref[...]
    acc = raw[0] + raw[1]
    nb = acc.shape[0]
    num = acc[:, :hc_in].reshape(nb, H, hc_in // H)
    den = acc[:, hc_in:hc_in + 8]
    xin = num / (den[:, :, None] + 1e-16)
    xin = xin.reshape(nb, hc_in) + b_ref[...][None, :]
    xin_ref[...] = jnp.where(xin > 0, xin, jnp.exp(xin) - 1.0)


@functools.partial(jax.jit, static_argnames=("hc_in",))
def _combine(raw, b, hc_in):
    row_in = hc_in + 16
    nrb = NR // 4
    return pl.pallas_call(
        functools.partial(_combine_body, hc_in),
        grid=(4,),
        in_specs=[
            pl.BlockSpec((2, nrb, row_in), lambda i: (0, i, 0)),
            pl.BlockSpec((hc_in,), lambda i: (0,)),
        ],
        out_specs=pl.BlockSpec((nrb, hc_in), lambda i: (i, 0)),
        out_shape=jax.ShapeDtypeStruct((NR, hc_in), jnp.float32),
    )(raw, b)


def _dense_body0(hc, x_ref, w_ref, as_ref, ad_ref, hext_ref, dstt_ref):
    n = x_ref.shape[0]
    h = jnp.dot(x_ref[...], w_ref[...], preferred_element_type=jnp.float32)
    asrc = jnp.dot(h, as_ref[...], preferred_element_type=jnp.float32)
    adst = jnp.dot(h, ad_ref[...], preferred_element_type=jnp.float32)
    hext_ref[...] = jnp.concatenate(
        [h, asrc, jnp.zeros((n, 8), jnp.float32)], axis=1)
    m = asrc.max(axis=0)[None, :] + adst
    mub = jnp.where(m > 0, m, 0.2 * m)
    dstt_ref[...] = jnp.concatenate([adst, mub], axis=1)


@functools.partial(jax.jit, static_argnames=("hc",))
def _dense0(x, W, As, Ad, hc):
    n = x.shape[0]
    row = hc + 16
    return pl.pallas_call(
        functools.partial(_dense_body0, hc),
        out_shape=(
            jax.ShapeDtypeStruct((n, row), jnp.float32),
            jax.ShapeDtypeStruct((n, 16), jnp.float32),
        ),
    )(x, W, As, Ad)


def _head_body(h_ref, batch_ref, f1w_ref, f1b_ref, f2w_ref, f2b_ref, out_ref):
    h = h_ref[...]
    gid = lax.broadcasted_iota(jnp.int32, (N, G), 1)
    P = (batch_ref[...] == gid).astype(jnp.float32)
    sums = lax.dot_general(P, h, (((0,), (0,)), ((), ())),
                           preferred_element_type=jnp.float32)
    cnt = jnp.sum(P, axis=0)
    gm = sums / jnp.maximum(cnt, 1.0)[:, None]
    z = jnp.dot(gm, f1w_ref[...], preferred_element_type=jnp.float32)
    z = jnp.maximum(z + f1b_ref[...][None, :], 0.0)
    out_ref[...] = (jnp.dot(z, f2w_ref[...], preferred_element_type=jnp.float32)
                    + f2b_ref[...][None, :])


@jax.jit
def _head(h, batch2, f1w, f1b, f2w, f2b):
    return pl.pallas_call(
        _head_body,
        out_shape=jax.ShapeDtypeStruct((G, 1), jnp.float32),
    )(h, batch2, f1w, f1b, f2w, f2b)


def _att_mat(a):
    Hh, C = a.shape
    return (jnp.eye(Hh, dtype=a.dtype)[:, None, :] * a[:, :, None]).reshape(Hh * C, Hh)


def _sc_edge_body(hc, idx_all, hext, dstt, out,
                  I0, I1, I2, I3, S0, Dt0, R0, S1, Dt1, R1, accum,
                  sS0, sD0, sS1, sD1, sI0, sI1, sI2, sI3, sW):
    row = hc + 16
    nv = hc // 16
    c = lax.axis_index("c")
    s = lax.axis_index("s")
    ncx = jnp.where(c == 0, NCH_A, NCH_B)
    cb = jnp.where(c == 0, s * NCH_A, NS * NCH_A + s * NCH_B)

    cph = hc // H
    lanes = lax.iota(jnp.int32, 16)
    idx_hi = lanes % 8 + 8
    mask_den = jnp.where(lanes < 8, 1.0, 0.0).astype(jnp.float32)

    Islots = (I0, I1, I2, I3)
    Isems = (sI0, sI1, sI2, sI3)
    data = ((S0, Dt0, R0, sS0, sD0), (S1, Dt1, R1, sS1, sD1))

    def _zrow(k, _):
        for j in range(row // 16):
            R0[k, pl.ds(16 * j, 16)] = jnp.zeros((16,), jnp.float32)
        return 0
    lax.fori_loop(0, KK, _zrow, 0)
    chunks = [KK] * (RPT // KK) + ([RPT % KK] if RPT % KK else [])
    off = 0
    for sz in chunks:
        pltpu.async_copy(R0.at[pl.ds(0, sz)],
                         accum.at[pl.ds(s * RPT + off, sz)], sW)
        off += sz
    off = 0
    for sz in chunks:
        pltpu.make_async_copy(R0.at[pl.ds(0, sz)],
                              accum.at[pl.ds(s * RPT + off, sz)], sW).wait()
        off += sz
    plsc.subcore_barrier()

    def _issue_idx(isl, g):
        pltpu.async_copy(idx_all.at[cb + g], Islots[isl], Isems[isl])

    def _issue_gather(isl, b):
        S, Dt, _, sS, sD = data[b]
        pltpu.async_copy(hext.at[Islots[isl].at[0]], S, sS)
        pltpu.async_copy(dstt.at[Islots[isl].at[1]], Dt, sD)

    for g in range(4):
        _issue_idx(g, g)
    for g in range(2):
        pltpu.make_async_copy(idx_all.at[cb + g], Islots[g], Isems[g]).wait()
        _issue_gather(g, g)

    def _quad(gq, _):
        for b4 in range(4):
            g = 4 * gq + b4
            b = b4 % 2
            isl = b4
            S, Dt, R, sS, sD = data[b]
            pltpu.make_async_copy(hext.at[Islots[isl].at[0]], S, sS).wait()
            pltpu.make_async_copy(dstt.at[Islots[isl].at[1]], Dt, sD).wait()

            @plsc.parallel_loop(0, KK, unroll=2)
            def _edge(k):
                sa = S[k, pl.ds(hc, 16)]
                dv = Dt[k, pl.ds(0, 16)]
                e16 = sa + dv
                lr = jnp.where(e16 > 0, e16, 0.2 * e16)
                mub16 = dv.at[idx_hi].get(mode="promise_in_bounds")
                wv = jnp.exp(lr - mub16)
                for j in range(nv):
                    if cph == 16:
                        idxj = lanes * 0 + j
                    else:
                        idxj = jnp.where(lanes < 8, 2 * j, 2 * j + 1)
                    wj = wv.at[idxj].get(mode="promise_in_bounds")
                    R[k, pl.ds(16 * j, 16)] = wj * S[k, pl.ds(16 * j, 16)]
                R[k, pl.ds(hc, 16)] = wv * mask_den

            pltpu.sync_copy(R, accum.at[Islots[isl].at[1]], add=True)

            @pl.when(g + 2 < ncx)
            def _():
                isl2 = (b4 + 2) % 4
                pltpu.make_async_copy(
                    idx_all.at[cb + g + 2], Islots[isl2], Isems[isl2]).wait()
                _issue_gather(isl2, b)

            @pl.when(g + 4 < ncx)
            def _():
                _issue_idx(isl, g + 4)
        return 0

    lax.fori_loop(0, ncx // 4, _quad, 0)
    plsc.subcore_barrier()

    off = 0
    for sz in chunks:
        r0 = s * RPT + off
        pltpu.async_copy(accum.at[pl.ds(r0, sz)], out.at[c, pl.ds(r0, sz)], sW)
        off += sz
    off = 0
    for sz in chunks:
        r0 = s * RPT + off
        pltpu.make_async_copy(accum.at[pl.ds(r0, sz)],
                              out.at[c, pl.ds(r0, sz)], sW).wait()
        off += sz


@functools.lru_cache(maxsize=None)
def _sc_edge_fn(hc):
    row = hc + 16
    mesh = plsc.VectorSubcoreMesh(core_axis_name="c", subcore_axis_name="s",
                                  num_cores=NC, num_subcores=NS)
    return pl.kernel(
        functools.partial(_sc_edge_body, hc),
        out_type=jax.ShapeDtypeStruct((NC, NR, row), jnp.float32),
        mesh=mesh,
        compiler_params=pltpu.CompilerParams(use_tc_tiling_on_sc=False),
        scratch_types=[
            pltpu.VMEM((2, KK), jnp.int32),
            pltpu.VMEM((2, KK), jnp.int32),
            pltpu.VMEM((2, KK), jnp.int32),
            pltpu.VMEM((2, KK), jnp.int32),
            pltpu.VMEM((KK, row), jnp.float32),
            pltpu.VMEM((KK, 16), jnp.float32),
            pltpu.VMEM((KK, row), jnp.float32),
            pltpu.VMEM((KK, row), jnp.float32),
            pltpu.VMEM((KK, 16), jnp.float32),
            pltpu.VMEM((KK, row), jnp.float32),
            pltpu.VMEM_SHARED((NR, row), jnp.float32),
            pltpu.SemaphoreType.DMA,
            pltpu.SemaphoreType.DMA,
            pltpu.SemaphoreType.DMA,
            pltpu.SemaphoreType.DMA,
            pltpu.SemaphoreType.DMA,
            pltpu.SemaphoreType.DMA,
            pltpu.SemaphoreType.DMA,
            pltpu.SemaphoreType.DMA,
            pltpu.SemaphoreType.DMA,
        ],
    )


def kernel(x, edge_index, batch, W1, a1s, a1d, b1, W2, a2s, a2d, b2,
           W3, a3s, a3d, b3, fc1_w, fc1_b, fc2_w, fc2_b):
    loop = jnp.arange(N, dtype=jnp.int32)
    fill = jnp.full((EPAD - E - N,), N, jnp.int32)
    src_pad = jnp.concatenate([edge_index[0].astype(jnp.int32), loop, fill])
    dst_pad = jnp.concatenate([edge_index[1].astype(jnp.int32), loop, fill])
    idx_all = jnp.stack([src_pad.reshape(TOTCH, KK),
                         dst_pad.reshape(TOTCH, KK)], axis=1)

    def tables(hext, dstt):
        hext = jnp.concatenate(
            [hext, jnp.zeros((1, hext.shape[1]), jnp.float32)], axis=0)
        dstt = jnp.concatenate([dstt, jnp.zeros((1, 16), jnp.float32)], axis=0)
        return hext, dstt

    hext, dstt = _dense0(x, W1, _att_mat(a1s), _att_mat(a1d), 64)
    raw = _sc_edge_fn(64)(idx_all, *tables(hext, dstt))
    xin = _combine(raw, b1, 64)
    hext, dstt = _dense0(xin, W2, _att_mat(a2s), _att_mat(a2d), 128)
    raw = _sc_edge_fn(128)(idx_all, hext, dstt)
    xin = _combine(raw, b2, 128)
    hext, dstt = _dense0(xin, W3, _att_mat(a3s), _att_mat(a3d), 128)
    raw = _sc_edge_fn(128)(idx_all, hext, dstt)
    h3 = _combine(raw, b3, 128)[:N]

    batch2 = batch.astype(jnp.int32).reshape(N, 1)
    return _head(h3, batch2, fc1_w, fc1_b, fc2_w, fc2_b)
